# baseline (speedup 1.0000x reference)
import functools

import jax
import jax.numpy as jnp
from jax.experimental import pallas as pl
from jax.experimental.pallas import tpu as pltpu

DIM = 768
HIDDEN = 3072
E = 8
TOPK = 2


def _router(x_ref, lnw_ref, lnb_ref, wr_ref, br_ref, t_ref, g_ref, stats_ref):
    xv = x_ref[...]
    mu = jnp.mean(xv, axis=1, keepdims=True)
    var = jnp.mean((xv - mu) ** 2, axis=1, keepdims=True)
    rf = (xv - mu) * jax.lax.rsqrt(var + 1e-5) * lnw_ref[...] + lnb_ref[...]
    t = jnp.maximum(t_ref[0, 0], 0.25)
    logits = (jnp.dot(rf, wr_ref[...], preferred_element_type=jnp.float32)
              + br_ref[...]) / t

    iota = jax.lax.broadcasted_iota(jnp.int32, logits.shape, 1)
    m1 = jnp.max(logits, axis=1, keepdims=True)
    i1 = jnp.min(jnp.where(logits == m1, iota, E), axis=1, keepdims=True)
    oh1 = iota == i1
    l2 = jnp.where(oh1, -jnp.inf, logits)
    m2 = jnp.max(l2, axis=1, keepdims=True)
    i2 = jnp.min(jnp.where(l2 == m2, iota, E), axis=1, keepdims=True)
    oh2 = iota == i2
    e2 = jnp.exp(m2 - m1)
    g1 = 1.0 / (1.0 + e2)
    g2 = e2 / (1.0 + e2)
    g_ref[...] = jnp.where(oh1, g1, 0.0) + jnp.where(oh2, g2, 0.0)

    pexp = jnp.exp(logits - m1)
    psum = jnp.sum(pexp, axis=1, keepdims=True)
    probs = pexp / psum
    lse = m1[:, 0] + jnp.log(psum[:, 0])
    router_z = jnp.mean(lse * lse)
    importance = jnp.mean(probs, axis=0)
    load_balance = jnp.mean((importance - 1.0 / E) ** 2)
    plogp = probs * jnp.log(jnp.maximum(probs, 1e-9))
    entropy = -jnp.mean(jnp.sum(plogp, axis=1))
    aux = load_balance + 0.001 * router_z - 0.001 * entropy
    stats_ref[...] = jnp.stack(
        [load_balance, router_z, entropy, aux] + [jnp.float32(0.0)] * 4
    )[None, :]


def _body(x_ref, lnw_ref, lnb_ref, wr_ref, br_ref, t_ref, b2_ref,
          w1_ref, b1_ref, w2_ref, out_ref, stats_ref, g_ref, xb_ref):
    e = pl.program_id(0)
    hb = pl.program_id(1)

    @pl.when((e == 0) & (hb == 0))
    def _():
        _router(x_ref, lnw_ref, lnb_ref, wr_ref, br_ref, t_ref,
                g_ref, stats_ref)
        xb_ref[...] = x_ref[...].astype(jnp.bfloat16)
        out_ref[...] = jnp.dot(g_ref[...], b2_ref[...],
                               preferred_element_type=jnp.float32)

    h = jnp.dot(xb_ref[...], w1_ref[0].astype(jnp.bfloat16),
                preferred_element_type=jnp.float32) + b1_ref[0]
    h = 0.5 * h * (1.0 + jax.lax.erf(h * 0.7071067811865476))
    contrib = jnp.dot(h.astype(jnp.bfloat16), w2_ref[0].astype(jnp.bfloat16),
                      preferred_element_type=jnp.float32)
    lane = jax.lax.broadcasted_iota(jnp.int32, (x_ref.shape[0], E), 1)
    g = jnp.sum(jnp.where(lane == e, g_ref[...], 0.0), axis=1, keepdims=True)
    out_ref[...] += g * contrib


@functools.partial(jax.jit, static_argnames=())
def kernel(x, ln_w, ln_b, Wr, br, temperature, W1, b1, W2, b2):
    B, S, D = x.shape
    T = B * S
    xf = x.reshape(T, D)
    HB = HIDDEN // 4

    out, stats = pl.pallas_call(
        _body,
        grid=(E, 4),
        in_specs=[
            pl.BlockSpec((T, D), lambda e, h: (0, 0)),
            pl.BlockSpec((1, D), lambda e, h: (0, 0)),
            pl.BlockSpec((1, D), lambda e, h: (0, 0)),
            pl.BlockSpec((D, E), lambda e, h: (0, 0)),
            pl.BlockSpec((1, E), lambda e, h: (0, 0)),
            pl.BlockSpec((1, 1), lambda e, h: (0, 0)),
            pl.BlockSpec((E, D), lambda e, h: (0, 0)),
            pl.BlockSpec((1, D, HB), lambda e, h: (e, 0, h)),
            pl.BlockSpec((1, 1, HB), lambda e, h: (e, 0, h)),
            pl.BlockSpec((1, HB, D), lambda e, h: (e, h, 0)),
        ],
        out_specs=(
            pl.BlockSpec((T, D), lambda e, h: (0, 0)),
            pl.BlockSpec((1, 8), lambda e, h: (0, 0)),
        ),
        out_shape=(
            jax.ShapeDtypeStruct((T, D), jnp.float32),
            jax.ShapeDtypeStruct((1, 8), jnp.float32),
        ),
        scratch_shapes=[
            pltpu.VMEM((T, E), jnp.float32),
            pltpu.VMEM((T, D), jnp.bfloat16),
        ],
    )(xf, ln_w.reshape(1, D), ln_b.reshape(1, D), Wr, br.reshape(1, E),
      temperature.reshape(1, 1).astype(jnp.float32), b2,
      W1, b1.reshape(E, 1, HIDDEN), W2)

    out = out.reshape(B, S, D)
    return (out, stats[0, 0], stats[0, 1], stats[0, 2], stats[0, 3])


# speedup vs baseline: 1.1630x; 1.1630x over previous
import functools

import jax
import jax.numpy as jnp
from jax.experimental import pallas as pl
from jax.experimental.pallas import tpu as pltpu

DIM = 768
HIDDEN = 3072
E = 8
TOPK = 2


def _router(x_ref, lnw_ref, lnb_ref, wr_ref, br_ref, t_ref, g_ref, stats_ref):
    xv = x_ref[...]
    mu = jnp.mean(xv, axis=1, keepdims=True)
    var = jnp.mean((xv - mu) ** 2, axis=1, keepdims=True)
    rf = (xv - mu) * jax.lax.rsqrt(var + 1e-5) * lnw_ref[...] + lnb_ref[...]
    t = jnp.maximum(t_ref[0, 0], 0.25)
    logits = (jnp.dot(rf, wr_ref[...], preferred_element_type=jnp.float32)
              + br_ref[...]) / t

    iota = jax.lax.broadcasted_iota(jnp.int32, logits.shape, 1)
    m1 = jnp.max(logits, axis=1, keepdims=True)
    i1 = jnp.min(jnp.where(logits == m1, iota, E), axis=1, keepdims=True)
    oh1 = iota == i1
    l2 = jnp.where(oh1, -jnp.inf, logits)
    m2 = jnp.max(l2, axis=1, keepdims=True)
    i2 = jnp.min(jnp.where(l2 == m2, iota, E), axis=1, keepdims=True)
    oh2 = iota == i2
    e2 = jnp.exp(m2 - m1)
    g1 = 1.0 / (1.0 + e2)
    g2 = e2 / (1.0 + e2)
    g_ref[...] = jnp.where(oh1, g1, 0.0) + jnp.where(oh2, g2, 0.0)

    pexp = jnp.exp(logits - m1)
    psum = jnp.sum(pexp, axis=1, keepdims=True)
    probs = pexp / psum
    lse = m1[:, 0] + jnp.log(psum[:, 0])
    router_z = jnp.mean(lse * lse)
    importance = jnp.mean(probs, axis=0)
    load_balance = jnp.mean((importance - 1.0 / E) ** 2)
    plogp = probs * jnp.log(jnp.maximum(probs, 1e-9))
    entropy = -jnp.mean(jnp.sum(plogp, axis=1))
    aux = load_balance + 0.001 * router_z - 0.001 * entropy
    stats_ref[...] = jnp.stack(
        [load_balance, router_z, entropy, aux] + [jnp.float32(0.0)] * 4
    )[None, :]


def _body(x_ref, lnw_ref, lnb_ref, wr_ref, br_ref, t_ref, b2_ref,
          w1_ref, b1_ref, w2_ref, out_ref, stats_ref, g_ref, xb_ref):
    e = pl.program_id(0)
    hb = pl.program_id(1)

    @pl.when((e == 0) & (hb == 0))
    def _():
        _router(x_ref, lnw_ref, lnb_ref, wr_ref, br_ref, t_ref,
                g_ref, stats_ref)
        xb_ref[...] = x_ref[...].astype(jnp.bfloat16)
        out_ref[...] = jnp.dot(g_ref[...], b2_ref[...],
                               preferred_element_type=jnp.float32)

    h = jnp.dot(xb_ref[...], w1_ref[0].astype(jnp.bfloat16),
                preferred_element_type=jnp.float32) + b1_ref[0]
    h = 0.5 * h * (1.0 + jax.lax.erf(h * 0.7071067811865476))
    contrib = jnp.dot(h.astype(jnp.bfloat16), w2_ref[0].astype(jnp.bfloat16),
                      preferred_element_type=jnp.float32)
    lane = jax.lax.broadcasted_iota(jnp.int32, (x_ref.shape[0], E), 1)
    g = jnp.sum(jnp.where(lane == e, g_ref[...], 0.0), axis=1, keepdims=True)
    out_ref[...] += g * contrib


@functools.partial(jax.jit, static_argnames=())
def kernel(x, ln_w, ln_b, Wr, br, temperature, W1, b1, W2, b2):
    B, S, D = x.shape
    T = B * S
    xf = x.reshape(T, D)
    HB = HIDDEN // 2

    out, stats = pl.pallas_call(
        _body,
        grid=(E, 2),
        in_specs=[
            pl.BlockSpec((T, D), lambda e, h: (0, 0)),
            pl.BlockSpec((1, D), lambda e, h: (0, 0)),
            pl.BlockSpec((1, D), lambda e, h: (0, 0)),
            pl.BlockSpec((D, E), lambda e, h: (0, 0)),
            pl.BlockSpec((1, E), lambda e, h: (0, 0)),
            pl.BlockSpec((1, 1), lambda e, h: (0, 0)),
            pl.BlockSpec((E, D), lambda e, h: (0, 0)),
            pl.BlockSpec((1, D, HB), lambda e, h: (e, 0, h)),
            pl.BlockSpec((1, 1, HB), lambda e, h: (e, 0, h)),
            pl.BlockSpec((1, HB, D), lambda e, h: (e, h, 0)),
        ],
        out_specs=(
            pl.BlockSpec((T, D), lambda e, h: (0, 0)),
            pl.BlockSpec((1, 8), lambda e, h: (0, 0)),
        ),
        out_shape=(
            jax.ShapeDtypeStruct((T, D), jnp.float32),
            jax.ShapeDtypeStruct((1, 8), jnp.float32),
        ),
        scratch_shapes=[
            pltpu.VMEM((T, E), jnp.float32),
            pltpu.VMEM((T, D), jnp.bfloat16),
        ],
    )(xf, ln_w.reshape(1, D), ln_b.reshape(1, D), Wr, br.reshape(1, E),
      temperature.reshape(1, 1).astype(jnp.float32), b2,
      W1, b1.reshape(E, 1, HIDDEN), W2)

    out = out.reshape(B, S, D)
    return (out, stats[0, 0], stats[0, 1], stats[0, 2], stats[0, 3])
